# Initial kernel scaffold; baseline (speedup 1.0000x reference)
import jax
import jax.numpy as jnp
from jax.experimental import pallas as pl


def kernel(x, edge_index, W1, b1, W2, b2, Wp1, bp1, Wp2, bp2, Wp3, bp3):
    raise NotImplementedError("write your pallas kernel here")


# speedup vs baseline: 3.2929x; 3.2929x over previous
import functools

import jax
import jax.numpy as jnp
from jax import lax
from jax.experimental import pallas as pl
from jax.experimental.pallas import tpu as pltpu
from jax.experimental.pallas import tpu_sc as plsc

N = 10000
E = 320000
D = 128

NC = 2
NS = 16
NW = NC * NS

CH = 80
EPW = E // NW
HPW = 2 * E // NW
NP = 10240
RPT = NP // NS
ZCH = 128


def _sc_mesh():
    return plsc.VectorSubcoreMesh(core_axis_name="c", subcore_axis_name="s")


@functools.cache
def _deg_kernel_fn():
    return functools.partial(
        pl.kernel,
        out_type=jax.ShapeDtypeStruct((NC, 2 * N), jnp.float32),
        mesh=_sc_mesh(),
        scratch_types=[
            pltpu.VMEM((CH,), jnp.int32),
            pltpu.VMEM((CH,), jnp.float32),
            pltpu.VMEM((2 * N,), jnp.float32),
            pltpu.VMEM_SHARED((2 * N,), jnp.float32),
        ],
    )(_deg_body)


def _deg_body(idx_hbm, out_hbm, idx_v, ones_v, big_v, hist_sh):
    c = lax.axis_index("c")
    s = lax.axis_index("s")
    wid = s * NC + c

    def fill_ones(i, _):
        ones_v[pl.ds(i * 16, 16)] = jnp.ones((16,), jnp.float32)
        return 0

    lax.fori_loop(0, CH // 16, fill_ones, 0)

    @pl.when(s == 0)
    def _zero():
        def z(i, _):
            big_v[pl.ds(i * 16, 16)] = jnp.zeros((16,), jnp.float32)
            return 0

        lax.fori_loop(0, (2 * N) // 16, z, 0)
        pltpu.sync_copy(big_v, hist_sh)

    plsc.subcore_barrier()

    base = wid * HPW

    def step(j, _):
        pltpu.sync_copy(idx_hbm.at[pl.ds(base + j * CH, CH)], idx_v)
        pltpu.sync_copy(ones_v, hist_sh.at[idx_v], add=True)
        return 0

    lax.fori_loop(0, HPW // CH, step, 0)
    plsc.subcore_barrier()

    @pl.when(s == 0)
    def _dump():
        pltpu.sync_copy(hist_sh, big_v)
        pltpu.sync_copy(big_v, out_hbm.at[c])


@functools.cache
def _spmm_kernel_fn():
    return functools.partial(
        pl.kernel,
        out_type=jax.ShapeDtypeStruct((NC, NP, D), jnp.float32),
        mesh=_sc_mesh(),
        scratch_types=[
            pltpu.VMEM((CH,), jnp.int32),
            pltpu.VMEM((CH,), jnp.int32),
            pltpu.VMEM((CH, D), jnp.float32),
            pltpu.VMEM((ZCH, D), jnp.float32),
            pltpu.VMEM_SHARED((NP, D), jnp.float32),
            pltpu.SemaphoreType.DMA,
        ],
    )(_spmm_body)


def _spmm_body(tab_hbm, src_hbm, dst_hbm, out_hbm, si_v, di_v, rows_v,
               blk_v, acc_sh, sem):
    c = lax.axis_index("c")
    s = lax.axis_index("s")
    wid = s * NC + c

    def zrow(i, _):
        r = i // (D // 16)
        k = i % (D // 16)
        blk_v[r, pl.ds(k * 16, 16)] = jnp.zeros((16,), jnp.float32)
        return 0

    lax.fori_loop(0, ZCH * (D // 16), zrow, 0)

    def zcopy(k, _):
        pltpu.sync_copy(blk_v, acc_sh.at[pl.ds(s * RPT + k * ZCH, ZCH)])
        return 0

    lax.fori_loop(0, RPT // ZCH, zcopy, 0)
    plsc.subcore_barrier()

    base = wid * EPW

    def step(j, _):
        pltpu.sync_copy(src_hbm.at[pl.ds(base + j * CH, CH)], si_v)
        pltpu.sync_copy(dst_hbm.at[pl.ds(base + j * CH, CH)], di_v)
        pltpu.async_copy(tab_hbm.at[si_v], rows_v, sem).wait()
        pltpu.sync_copy(rows_v, acc_sh.at[di_v], add=True)
        return 0

    lax.fori_loop(0, EPW // CH, step, 0)
    plsc.subcore_barrier()

    def dump(k, _):
        r0 = s * RPT + k * ZCH
        pltpu.sync_copy(acc_sh.at[pl.ds(r0, ZCH)], blk_v)
        pltpu.sync_copy(blk_v, out_hbm.at[c, pl.ds(r0, ZCH)])
        return 0

    lax.fori_loop(0, RPT // ZCH, dump, 0)


@functools.cache
def _edge_gather_kernel_fn():
    return functools.partial(
        pl.kernel,
        out_type=(
            jax.ShapeDtypeStruct((E, D), jnp.float32),
            jax.ShapeDtypeStruct((E, D), jnp.float32),
        ),
        mesh=_sc_mesh(),
        scratch_types=[
            pltpu.VMEM((CH,), jnp.int32),
            pltpu.VMEM((CH,), jnp.int32),
            pltpu.VMEM((CH, D), jnp.float32),
            pltpu.VMEM((CH, D), jnp.float32),
            pltpu.SemaphoreType.DMA,
            pltpu.SemaphoreType.DMA,
        ],
    )(_edge_gather_body)


def _edge_gather_body(a_hbm, b_hbm, src_hbm, dst_hbm, oa_hbm, ob_hbm,
                      si_v, di_v, ra_v, rb_v, sema, semb):
    c = lax.axis_index("c")
    s = lax.axis_index("s")
    wid = s * NC + c
    base = wid * EPW

    def step(j, _):
        e0 = base + j * CH
        pltpu.sync_copy(src_hbm.at[pl.ds(e0, CH)], si_v)
        pltpu.sync_copy(dst_hbm.at[pl.ds(e0, CH)], di_v)
        ca = pltpu.async_copy(a_hbm.at[si_v], ra_v, sema)
        cb = pltpu.async_copy(b_hbm.at[di_v], rb_v, semb)
        ca.wait()
        pltpu.sync_copy(ra_v, oa_hbm.at[pl.ds(e0, CH)])
        cb.wait()
        pltpu.sync_copy(rb_v, ob_hbm.at[pl.ds(e0, CH)])
        return 0

    lax.fori_loop(0, EPW // CH, step, 0)


BN = 2000


def _prep_body(dgo_ref, dgi_ref, x_ref, t_ref, no_ref, ni_ref):
    no = lax.rsqrt(jnp.maximum(dgo_ref[...], 1.0))
    ni = lax.rsqrt(jnp.maximum(dgi_ref[...], 1.0))
    no_ref[...] = no
    ni_ref[...] = ni
    t_ref[...] = x_ref[...] * no


def _prep_tc(dgo, dgi, x):
    return pl.pallas_call(
        _prep_body,
        grid=(N // BN,),
        in_specs=[
            pl.BlockSpec((BN, 1), lambda i: (i, 0)),
            pl.BlockSpec((BN, 1), lambda i: (i, 0)),
            pl.BlockSpec((BN, D), lambda i: (i, 0)),
        ],
        out_specs=[
            pl.BlockSpec((BN, D), lambda i: (i, 0)),
            pl.BlockSpec((BN, 1), lambda i: (i, 0)),
            pl.BlockSpec((BN, 1), lambda i: (i, 0)),
        ],
        out_shape=[
            jax.ShapeDtypeStruct((N, D), jnp.float32),
            jax.ShapeDtypeStruct((N, 1), jnp.float32),
            jax.ShapeDtypeStruct((N, 1), jnp.float32),
        ],
    )(dgo, dgi, x)


def _layer_body(a_ref, ni_ref, no_ref, w_ref, b_ref, t_ref, *, act, scale):
    agg = (a_ref[0] + a_ref[1]) * ni_ref[...]
    h = jnp.dot(agg, w_ref[...], preferred_element_type=jnp.float32) + b_ref[...]
    if act:
        h = jnp.maximum(h, 0.0)
    if scale:
        h = h * no_ref[...]
    t_ref[...] = h


def _layer_tc(aggp, ni, no, w, b, act, scale):
    body = functools.partial(_layer_body, act=act, scale=scale)
    return pl.pallas_call(
        body,
        grid=(N // BN,),
        in_specs=[
            pl.BlockSpec((2, BN, D), lambda i: (0, i, 0)),
            pl.BlockSpec((BN, 1), lambda i: (i, 0)),
            pl.BlockSpec((BN, 1), lambda i: (i, 0)),
            pl.BlockSpec((D, D), lambda i: (0, 0)),
            pl.BlockSpec((1, D), lambda i: (0, 0)),
        ],
        out_specs=pl.BlockSpec((BN, D), lambda i: (i, 0)),
        out_shape=jax.ShapeDtypeStruct((N, D), jnp.float32),
    )(aggp, ni, no, w, b)


def _head_body(h_ref, wa_ref, wb_ref, bp_ref, a_ref, b_ref):
    h = h_ref[...]
    a_ref[...] = jnp.dot(h, wa_ref[...], preferred_element_type=jnp.float32)
    b_ref[...] = (jnp.dot(h, wb_ref[...], preferred_element_type=jnp.float32)
                  + bp_ref[...])


def _head_tc(h, wp1a, wp1b, bp1):
    return pl.pallas_call(
        _head_body,
        grid=(N // BN,),
        in_specs=[
            pl.BlockSpec((BN, D), lambda i: (i, 0)),
            pl.BlockSpec((D, D), lambda i: (0, 0)),
            pl.BlockSpec((D, D), lambda i: (0, 0)),
            pl.BlockSpec((1, D), lambda i: (0, 0)),
        ],
        out_specs=[
            pl.BlockSpec((BN, D), lambda i: (i, 0)),
            pl.BlockSpec((BN, D), lambda i: (i, 0)),
        ],
        out_shape=[
            jax.ShapeDtypeStruct((N, D), jnp.float32),
            jax.ShapeDtypeStruct((N, D), jnp.float32),
        ],
    )(h, wp1a, wp1b, bp1)


BE = 4000


def _mlp_body(as_ref, bs_ref, w2_ref, b2_ref, w3_ref, b3_ref, o_ref):
    z1 = jnp.maximum(as_ref[...] + bs_ref[...], 0.0)
    z2 = jnp.dot(z1, w2_ref[...], preferred_element_type=jnp.float32) + b2_ref[...]
    z2 = jnp.maximum(z2, 0.0)
    z3 = jnp.dot(z2, w3_ref[...], preferred_element_type=jnp.float32) + b3_ref[...]
    o_ref[...] = jax.nn.sigmoid(z3)


def _mlp_tc(asrc, bdst, wp2, bp2, wp3, bp3):
    return pl.pallas_call(
        _mlp_body,
        grid=(E // BE,),
        in_specs=[
            pl.BlockSpec((BE, D), lambda i: (i, 0)),
            pl.BlockSpec((BE, D), lambda i: (i, 0)),
            pl.BlockSpec((D, 64), lambda i: (0, 0)),
            pl.BlockSpec((1, 64), lambda i: (0, 0)),
            pl.BlockSpec((64, 1), lambda i: (0, 0)),
            pl.BlockSpec((1, 1), lambda i: (0, 0)),
        ],
        out_specs=pl.BlockSpec((BE, 1), lambda i: (i, 0)),
        out_shape=jax.ShapeDtypeStruct((E, 1), jnp.float32),
    )(asrc, bdst, wp2, bp2, wp3, bp3)


def kernel(x, edge_index, W1, b1, W2, b2, Wp1, bp1, Wp2, bp2, Wp3, bp3):
    ei = edge_index.astype(jnp.int32)
    src = ei[0]
    dst = ei[1]
    hist_idx = jnp.concatenate([src, dst + N])

    degp = _deg_kernel_fn()(hist_idx)
    deg = degp[0] + degp[1]
    t0, no, ni = _prep_tc(deg[:N].reshape(N, 1), deg[N:].reshape(N, 1), x)

    agg1p = _spmm_kernel_fn()(t0, src, dst)
    t1 = _layer_tc(agg1p, ni, no, W1, b1.reshape(1, D), act=True, scale=True)

    agg2p = _spmm_kernel_fn()(t1, src, dst)
    h2 = _layer_tc(agg2p, ni, no, W2, b2.reshape(1, D), act=False, scale=False)

    atab, btab = _head_tc(h2, Wp1[:D], Wp1[D:], bp1.reshape(1, D))
    asrc, bdst = _edge_gather_kernel_fn()(atab, btab, src, dst)

    return _mlp_tc(asrc, bdst, Wp2, bp2.reshape(1, 64), Wp3, bp3.reshape(1, 1))
